# baseline (speedup 1.0000x reference)
import functools

import jax
import jax.numpy as jnp
from jax import lax
from jax.experimental import pallas as pl
from jax.experimental.pallas import tpu as pltpu
from jax.experimental.pallas import tpu_sc as plsc

_NC = 2
_NS = 16
_NW = _NC * _NS
_L = 16

_CHUNK = 64


def _floorf(x):
    xi = x.astype(jnp.int32)
    xf = xi.astype(jnp.float32)
    return jnp.where(xf > x, xf - 1.0, xf), jnp.where(xf > x, xi - 1, xi)


def _warp_sc(imgT, fx, fy, *, N, C, H, W, NB):
    HW = H * W
    per_w = N // _NW
    chunks = per_w // _CHUNK
    cpr = W // _CHUNK

    @functools.partial(
        pl.kernel,
        out_type=jax.ShapeDtypeStruct((N, C), jnp.float32),
        mesh=plsc.VectorSubcoreMesh(core_axis_name="c", subcore_axis_name="s"),
        compiler_params=pltpu.CompilerParams(use_tc_tiling_on_sc=False),
        scratch_types=[
            pltpu.VMEM((_CHUNK,), jnp.float32),
            pltpu.VMEM((_CHUNK,), jnp.float32),
            pltpu.VMEM((_CHUNK,), jnp.int32),
            pltpu.VMEM((_CHUNK,), jnp.int32),
            pltpu.VMEM((_CHUNK,), jnp.int32),
            pltpu.VMEM((_CHUNK,), jnp.int32),
            pltpu.VMEM((_CHUNK,), jnp.float32),
            pltpu.VMEM((_CHUNK,), jnp.float32),
            pltpu.VMEM((_CHUNK,), jnp.float32),
            pltpu.VMEM((_CHUNK,), jnp.float32),
            pltpu.VMEM((_CHUNK, C), jnp.float32),
            pltpu.VMEM((_CHUNK, C), jnp.float32),
            pltpu.VMEM((_CHUNK, C), jnp.float32),
            pltpu.VMEM((_CHUNK, C), jnp.float32),
            pltpu.VMEM((_CHUNK, C), jnp.float32),
            pltpu.SemaphoreType.DMA,
        ],
    )
    def k(imgT_hbm, fx_hbm, fy_hbm, out_hbm, fx_v, fy_v, ia_v, ib_v, ic_v,
          id_v, wa_v, wb_v, wc_v, wd_v, ra_v, rb_v, rc_v, rd_v, out_v, sem):
        wid = lax.axis_index("s") * _NC + lax.axis_index("c")
        batch = wid // (_NW // NB)
        roff = batch * HW

        def chunk_body(g, carry):
            base = wid * per_w + g * _CHUNK
            gy = (base // W) % H
            gx0 = base % W
            pltpu.sync_copy(fx_hbm.at[pl.ds(base, _CHUNK)], fx_v)
            pltpu.sync_copy(fy_hbm.at[pl.ds(base, _CHUNK)], fy_v)
            maxx = jnp.float32(W - 1)
            maxy = jnp.float32(H - 1)
            for i in range(_CHUNK // _L):
                sl = pl.ds(i * _L, _L)
                lane = lax.iota(jnp.int32, _L).astype(jnp.float32)
                posx = lane + (jnp.float32(gx0 + i * _L) + fx_v[sl])
                posy = jnp.float32(gy) + fy_v[sl]
                posx = jnp.minimum(jnp.maximum(posx, -1.0), jnp.float32(W))
                posy = jnp.minimum(jnp.maximum(posy, -1.0), jnp.float32(H))
                x0f, x0i = _floorf(posx)
                y0f, y0i = _floorf(posy)
                xw = posx - x0f
                yw = posy - y0f
                x0c = jnp.minimum(jnp.maximum(x0i, 0), jnp.int32(W - 1))
                x1c = jnp.minimum(jnp.maximum(x0i + 1, 0), jnp.int32(W - 1))
                y0c = jnp.minimum(jnp.maximum(y0i, 0), jnp.int32(H - 1))
                y1c = jnp.minimum(jnp.maximum(y0i + 1, 0), jnp.int32(H - 1))
                b0 = roff + y0c * W
                b1 = roff + y1c * W
                ia_v[sl] = b0 + x0c
                ib_v[sl] = b1 + x0c
                ic_v[sl] = b0 + x1c
                id_v[sl] = b1 + x1c
                wa_v[sl] = (1.0 - xw) * (1.0 - yw)
                wb_v[sl] = (1.0 - xw) * yw
                wc_v[sl] = xw * (1.0 - yw)
                wd_v[sl] = xw * yw
            cps = [
                pltpu.async_copy(imgT_hbm.at[ia_v], ra_v, sem),
                pltpu.async_copy(imgT_hbm.at[ib_v], rb_v, sem),
                pltpu.async_copy(imgT_hbm.at[ic_v], rc_v, sem),
                pltpu.async_copy(imgT_hbm.at[id_v], rd_v, sem),
            ]
            for cp in cps:
                cp.wait()

            for grp in range(_CHUNK // _L):
                gsl = pl.ds(grp * _L, _L)
                wa16 = wa_v[gsl]
                wb16 = wb_v[gsl]
                wc16 = wc_v[gsl]
                wd16 = wd_v[gsl]
                for lanej in range(_L):
                    j = grp * _L + lanej
                    was = wa16[lanej]
                    wbs = wb16[lanej]
                    wcs = wc16[lanej]
                    wds = wd16[lanej]
                    for cg in range(C // _L):
                        cs = pl.ds(cg * _L, _L)
                        out_v[j, cs] = (was * ra_v[j, cs] + wbs * rb_v[j, cs]
                                        + wcs * rc_v[j, cs] + wds * rd_v[j, cs])
            pltpu.sync_copy(out_v, out_hbm.at[pl.ds(base, _CHUNK)])
            return carry

        lax.fori_loop(0, chunks, chunk_body, 0)

    return k(imgT, fx, fy)


def kernel(img, flo):
    B, C, H, W = img.shape
    N = B * H * W
    imgT = jnp.transpose(img, (0, 2, 3, 1)).reshape(N, C)
    fx = flo[:, 0].reshape(N)
    fy = flo[:, 1].reshape(N)
    outT = _warp_sc(imgT, fx, fy, N=N, C=C, H=H, W=W, NB=B)
    return jnp.transpose(outT.reshape(B, H, W, C), (0, 3, 1, 2))


# speedup vs baseline: 1.8234x; 1.8234x over previous
import functools

import jax
import jax.numpy as jnp
from jax import lax
from jax.experimental import pallas as pl
from jax.experimental.pallas import tpu as pltpu
from jax.experimental.pallas import tpu_sc as plsc

_NC = 2
_NS = 16
_NW = _NC * _NS
_L = 16

_K = 64


def _floorf(x):
    xi = x.astype(jnp.int32)
    xf = xi.astype(jnp.float32)
    return jnp.where(xf > x, xf - 1.0, xf), jnp.where(xf > x, xi - 1, xi)


def _warp_sc(imgT, fx, fy, *, N, C, H, W, NB):
    HW = H * W
    per_w = N // _NW
    chunks = per_w // _K
    nh = chunks // 2

    @functools.partial(
        pl.kernel,
        out_type=jax.ShapeDtypeStruct((N, C), jnp.float32),
        mesh=plsc.VectorSubcoreMesh(core_axis_name="c", subcore_axis_name="s"),
        compiler_params=pltpu.CompilerParams(use_tc_tiling_on_sc=False),
        scratch_types=[
            pltpu.VMEM((2, _K), jnp.float32),
            pltpu.VMEM((2, _K), jnp.float32),
            pltpu.VMEM((2, 4, _K), jnp.int32),
            pltpu.VMEM((2, 4, _K), jnp.float32),
            pltpu.VMEM((2, 4, _K, C), jnp.float32),
            pltpu.VMEM((2, _K, C), jnp.float32),
            pltpu.SemaphoreType.DMA,
            pltpu.SemaphoreType.DMA,
            pltpu.SemaphoreType.DMA,
            pltpu.SemaphoreType.DMA,
            pltpu.SemaphoreType.DMA,
            pltpu.SemaphoreType.DMA,
        ],
    )
    def k(imgT_hbm, fx_hbm, fy_hbm, out_hbm, f_x, f_y, idx, wgt, rows, orows,
          sem_fe, sem_fo, sem_ge, sem_go, sem_oe, sem_oo):
        wid = lax.axis_index("s") * _NC + lax.axis_index("c")
        batch = wid // (_NW // NB)
        roff = batch * HW
        start0 = wid * per_w
        sem_f = (sem_fe, sem_fo)
        sem_g = (sem_ge, sem_go)
        sem_o = (sem_oe, sem_oo)

        def issue_f(base, par):
            pltpu.async_copy(fx_hbm.at[pl.ds(base, _K)], f_x.at[par], sem_f[par])
            pltpu.async_copy(fy_hbm.at[pl.ds(base, _K)], f_y.at[par], sem_f[par])

        def wait_f(par):
            pltpu.make_async_copy(
                fx_hbm.at[pl.ds(0, _K)], f_x.at[par], sem_f[par]).wait()
            pltpu.make_async_copy(
                fy_hbm.at[pl.ds(0, _K)], f_y.at[par], sem_f[par]).wait()

        def compute_iw(base, par):
            gy = (base // W) % H
            gx0 = base % W
            for i in range(_K // _L):
                sl = pl.ds(i * _L, _L)
                lane = lax.iota(jnp.int32, _L).astype(jnp.float32)
                posx = lane + (jnp.float32(gx0 + i * _L) + f_x[par, sl])
                posy = jnp.float32(gy) + f_y[par, sl]
                posx = jnp.minimum(jnp.maximum(posx, -1.0), jnp.float32(W))
                posy = jnp.minimum(jnp.maximum(posy, -1.0), jnp.float32(H))
                x0f, x0i = _floorf(posx)
                y0f, y0i = _floorf(posy)
                xw = posx - x0f
                yw = posy - y0f
                x0c = jnp.minimum(jnp.maximum(x0i, 0), jnp.int32(W - 1))
                x1c = jnp.minimum(jnp.maximum(x0i + 1, 0), jnp.int32(W - 1))
                y0c = jnp.minimum(jnp.maximum(y0i, 0), jnp.int32(H - 1))
                y1c = jnp.minimum(jnp.maximum(y0i + 1, 0), jnp.int32(H - 1))
                b0 = roff + y0c * W
                b1 = roff + y1c * W
                idx[par, 0, sl] = b0 + x0c
                idx[par, 1, sl] = b1 + x0c
                idx[par, 2, sl] = b0 + x1c
                idx[par, 3, sl] = b1 + x1c
                wgt[par, 0, sl] = (1.0 - xw) * (1.0 - yw)
                wgt[par, 1, sl] = (1.0 - xw) * yw
                wgt[par, 2, sl] = xw * (1.0 - yw)
                wgt[par, 3, sl] = xw * yw

        def issue_g(par):
            for t in range(4):
                pltpu.async_copy(
                    imgT_hbm.at[idx.at[par, t]], rows.at[par, t], sem_g[par])

        def wait_g(par):
            for t in range(4):
                pltpu.make_async_copy(
                    imgT_hbm.at[idx.at[par, t]], rows.at[par, t],
                    sem_g[par]).wait()

        def combine(par):
            def grp_body(grp, carry):
                gsl = pl.ds(grp * _L, _L)
                wa16 = wgt[par, 0, gsl]
                wb16 = wgt[par, 1, gsl]
                wc16 = wgt[par, 2, gsl]
                wd16 = wgt[par, 3, gsl]
                for lanej in range(_L):
                    j = grp * _L + lanej
                    was = wa16[lanej]
                    wbs = wb16[lanej]
                    wcs = wc16[lanej]
                    wds = wd16[lanej]
                    for cg in range(C // _L):
                        cs = pl.ds(cg * _L, _L)
                        orows[par, j, cs] = (
                            was * rows[par, 0, j, cs]
                            + wbs * rows[par, 1, j, cs]
                            + wcs * rows[par, 2, j, cs]
                            + wds * rows[par, 3, j, cs])
                return carry

            lax.fori_loop(0, _K // _L, grp_body, 0)

        def issue_o(base, par):
            pltpu.async_copy(
                orows.at[par], out_hbm.at[pl.ds(base, _K)], sem_o[par])

        def wait_o(par):
            pltpu.make_async_copy(
                orows.at[par], out_hbm.at[pl.ds(0, _K)], sem_o[par]).wait()

        issue_f(start0, 0)
        issue_f(start0 + _K, 1)
        wait_f(0)
        compute_iw(start0, 0)
        issue_g(0)

        def body(t, carry):
            g0 = start0 + t * (2 * _K)
            wait_f(1)
            compute_iw(g0 + _K, 1)
            issue_g(1)

            @pl.when(t < nh - 1)
            def _():
                issue_f(g0 + 2 * _K, 0)

            wait_g(0)

            @pl.when(t > 0)
            def _():
                wait_o(0)

            combine(0)
            issue_o(g0, 0)

            @pl.when(t < nh - 1)
            def _():
                wait_f(0)
                compute_iw(g0 + 2 * _K, 0)
                issue_g(0)
                issue_f(g0 + 3 * _K, 1)

            wait_g(1)

            @pl.when(t > 0)
            def _():
                wait_o(1)

            combine(1)
            issue_o(g0 + _K, 1)
            return carry

        lax.fori_loop(0, nh, body, 0)
        wait_o(0)
        wait_o(1)

    return k(imgT, fx, fy)


def kernel(img, flo):
    B, C, H, W = img.shape
    N = B * H * W
    imgT = jnp.transpose(img, (0, 2, 3, 1)).reshape(N, C)
    fx = flo[:, 0].reshape(N)
    fy = flo[:, 1].reshape(N)
    outT = _warp_sc(imgT, fx, fy, N=N, C=C, H=H, W=W, NB=B)
    return jnp.transpose(outT.reshape(B, H, W, C), (0, 3, 1, 2))
